# baseline (speedup 1.0000x reference)
import functools

import jax
import jax.numpy as jnp
from jax import lax
from jax.experimental import pallas as pl
from jax.experimental.pallas import tpu as pltpu
from jax.experimental.pallas import tpu_sc as plsc

B = 4096
CONT_DIM = 128
DIM_USER = 32
DIM_ITEM = 32
DIM_CAT = 16
VOCAB_USER = 100000
VOCAB_ITEM = 100000
VOCAB_CAT = 1000

_NC, _NS = 2, 16
_NW = _NC * _NS
_L = 16


@functools.lru_cache(maxsize=1)
def _make_gather3():
    mesh = plsc.VectorSubcoreMesh(
        core_axis_name="c", subcore_axis_name="s",
        num_cores=_NC, num_subcores=_NS)

    @functools.partial(
        pl.kernel,
        mesh=mesh,
        out_type=(
            jax.ShapeDtypeStruct((DIM_USER, B), jnp.float32),
            jax.ShapeDtypeStruct((DIM_ITEM, B), jnp.float32),
            jax.ShapeDtypeStruct((DIM_CAT, B), jnp.float32),
        ),
        scratch_types=[
            pltpu.VMEM((B,), jnp.int32),
            pltpu.VMEM((B,), jnp.int32),
            pltpu.VMEM((B,), jnp.int32),
            pltpu.VMEM((B,), jnp.float32),
            pltpu.VMEM((VOCAB_USER,), jnp.float32),
            pltpu.VMEM((VOCAB_CAT,), jnp.float32),
            pltpu.SemaphoreType.DMA,
            pltpu.SemaphoreType.DMA,
            pltpu.SemaphoreType.DMA,
            pltpu.SemaphoreType.DMA,
            pltpu.SemaphoreType.DMA,
        ],
        compiler_params=pltpu.CompilerParams(needs_layout_passes=False),
    )
    def _gather3(uid_hbm, iid_hbm, cid_hbm, tab_u, tab_i, tab_c,
                 out_u, out_i, out_c, idx_u, idx_i, idx_c, res_v,
                 row_v, rowc_v, sem_iu, sem_ii, sem_ic, sem_r, sem_c):
        wid = lax.axis_index("s") * _NC + lax.axis_index("c")
        tf = wid >> 3
        fs = wid & 7

        def gather_loop(ids, row_buf, out_row_hbm):
            @pl.loop(0, B, step=_L, unroll=8)
            def _(c):
                iv = ids[pl.ds(c, _L)]
                res_v[pl.ds(c, _L)] = plsc.load_gather(row_buf, [iv])

            pltpu.sync_copy(res_v, out_row_hbm)

        c_iu = pltpu.async_copy(uid_hbm, idx_u, sem_iu)
        c_ii = pltpu.async_copy(iid_hbm, idx_i, sem_ii)
        c_ic = pltpu.async_copy(cid_hbm, idx_c, sem_ic)
        c_ru = pltpu.async_copy(tab_u.at[tf, fs], row_v, sem_r)
        is_cat = wid < DIM_CAT

        @pl.when(is_cat)
        def _():
            pltpu.async_copy(tab_c.at[wid >> 3, wid & 7], rowc_v, sem_c).wait()

        c_ru.wait()
        c_iu.wait()
        gather_loop(idx_u, row_v, out_u.at[wid])
        c_ri = pltpu.async_copy(tab_i.at[tf, fs], row_v, sem_r)

        @pl.when(is_cat)
        def _():
            c_ic.wait()
            gather_loop(idx_c, rowc_v, out_c.at[wid])

        @pl.when(jnp.logical_not(is_cat))
        def _():
            c_ic.wait()

        c_ri.wait()
        c_ii.wait()
        gather_loop(idx_i, row_v, out_i.at[wid])

    return _gather3


def _bn_relu(x, g, be, eps=1e-5):
    n = x.shape[0]
    ones = jnp.ones((1, n), dtype=jnp.float32)
    m = jnp.dot(ones, x, preferred_element_type=jnp.float32) / n
    e2 = jnp.dot(ones, x * x, preferred_element_type=jnp.float32) / n
    v = e2 - m * m
    a = g * jax.lax.rsqrt(v + eps)
    b = be - m * a
    return jnp.maximum(x * a + b, 0.0)


def _dot(a, b):
    bf16 = jnp.bfloat16
    return jnp.dot(a.astype(bf16), b.astype(bf16),
                   preferred_element_type=jnp.float32)


def _dot_t(a, b):
    bf16 = jnp.bfloat16
    return lax.dot_general(a.astype(bf16), b.astype(bf16),
                           (((0,), (0,)), ((), ())),
                           preferred_element_type=jnp.float32)


def _cont_body(cont_ref, w1a_ref, b1_ref, p1_ref):
    p1_ref[...] = _dot(cont_ref[...], w1a_ref[...]) + b1_ref[...]


def _mlp_body(p1_ref, eu_ref, ei_ref, ec_ref,
              w1b_ref, w1c_ref, w1d_ref,
              w2_ref, b2_ref, w3_ref, b3_ref, w4_ref, b4_ref,
              g1_ref, be1_ref, g2_ref, be2_ref, g3_ref, be3_ref,
              out_ref):
    h = (p1_ref[...]
         + _dot_t(eu_ref[...], w1b_ref[...])
         + _dot_t(ei_ref[...], w1c_ref[...])
         + _dot_t(ec_ref[...], w1d_ref[...]))
    h = _bn_relu(h, g1_ref[...], be1_ref[...])
    h = _dot(h, w2_ref[...]) + b2_ref[...]
    h = _bn_relu(h, g2_ref[...], be2_ref[...])
    h = _dot(h, w3_ref[...]) + b3_ref[...]
    h = _bn_relu(h, g3_ref[...], be3_ref[...])
    out_ref[...] = _dot(h, w4_ref[...]) + b4_ref[...]


def kernel(continuous_features, user_id, item_id, category,
           emb_user, emb_item, emb_cat,
           W1, b1, W2, b2, W3, b3, W4, b4,
           g1, be1, g2, be2, g3, be3):
    euT, eiT, ecT = _make_gather3()(
        user_id.astype(jnp.int32), item_id.astype(jnp.int32),
        category.astype(jnp.int32),
        emb_user.T.reshape(DIM_USER // 8, 8, VOCAB_USER),
        emb_item.T.reshape(DIM_ITEM // 8, 8, VOCAB_ITEM),
        emb_cat.T.reshape(DIM_CAT // 8, 8, VOCAB_CAT))

    w1a = W1[:CONT_DIM]
    w1b = W1[CONT_DIM:CONT_DIM + DIM_USER]
    w1c = W1[CONT_DIM + DIM_USER:CONT_DIM + DIM_USER + DIM_ITEM]
    w1d = W1[CONT_DIM + DIM_USER + DIM_ITEM:]

    cont_mm = pl.pallas_call(
        _cont_body,
        out_shape=jax.ShapeDtypeStruct((B, W1.shape[1]), jnp.float32),
    )
    p1 = cont_mm(continuous_features, w1a, b1.reshape(1, -1))

    mlp = pl.pallas_call(
        _mlp_body,
        out_shape=jax.ShapeDtypeStruct((B, W4.shape[1]), jnp.float32),
    )
    return mlp(p1, euT, eiT, ecT,
               w1b, w1c, w1d,
               W2, b2.reshape(1, -1), W3, b3.reshape(1, -1),
               W4, b4.reshape(1, -1),
               g1.reshape(1, -1), be1.reshape(1, -1),
               g2.reshape(1, -1), be2.reshape(1, -1),
               g3.reshape(1, -1), be3.reshape(1, -1))


# speedup vs baseline: 1.0619x; 1.0619x over previous
import functools

import jax
import jax.numpy as jnp
from jax import lax
from jax.experimental import pallas as pl
from jax.experimental.pallas import tpu as pltpu
from jax.experimental.pallas import tpu_sc as plsc

B = 4096
CONT_DIM = 128
DIM_USER = 32
DIM_ITEM = 32
DIM_CAT = 16
VOCAB_USER = 100000
VOCAB_ITEM = 100000
VOCAB_CAT = 1000

_NC, _NS = 2, 16
_NW = _NC * _NS
_L = 16


@functools.lru_cache(maxsize=1)
def _make_gather3():
    mesh = plsc.VectorSubcoreMesh(
        core_axis_name="c", subcore_axis_name="s",
        num_cores=_NC, num_subcores=_NS)

    @functools.partial(
        pl.kernel,
        mesh=mesh,
        out_type=(
            jax.ShapeDtypeStruct((DIM_USER, B), jnp.float32),
            jax.ShapeDtypeStruct((DIM_ITEM, B), jnp.float32),
            jax.ShapeDtypeStruct((DIM_CAT, B), jnp.float32),
        ),
        scratch_types=[
            pltpu.VMEM((B,), jnp.int32),
            pltpu.VMEM((B,), jnp.int32),
            pltpu.VMEM((B,), jnp.int32),
            pltpu.VMEM((B,), jnp.float32),
            pltpu.VMEM((VOCAB_USER,), jnp.float32),
            pltpu.VMEM((VOCAB_CAT,), jnp.float32),
            pltpu.SemaphoreType.DMA,
            pltpu.SemaphoreType.DMA,
            pltpu.SemaphoreType.DMA,
            pltpu.SemaphoreType.DMA,
            pltpu.SemaphoreType.DMA,
        ],
        compiler_params=pltpu.CompilerParams(needs_layout_passes=False),
    )
    def _gather3(uid_hbm, iid_hbm, cid_hbm, tab_u, tab_i, tab_c,
                 out_u, out_i, out_c, idx_u, idx_i, idx_c, res_v,
                 row_v, rowc_v, sem_iu, sem_ii, sem_ic, sem_r, sem_c):
        wid = lax.axis_index("s") * _NC + lax.axis_index("c")
        tf = wid >> 3
        fs = wid & 7

        def gather_loop(ids, row_buf, out_row_hbm):
            @pl.loop(0, B, step=_L, unroll=8)
            def _(c):
                iv = ids[pl.ds(c, _L)]
                res_v[pl.ds(c, _L)] = plsc.load_gather(row_buf, [iv])

            pltpu.sync_copy(res_v, out_row_hbm)

        c_iu = pltpu.async_copy(uid_hbm, idx_u, sem_iu)
        c_ii = pltpu.async_copy(iid_hbm, idx_i, sem_ii)
        c_ic = pltpu.async_copy(cid_hbm, idx_c, sem_ic)
        c_ru = pltpu.async_copy(tab_u.at[tf, fs], row_v, sem_r)
        is_cat = wid < DIM_CAT

        @pl.when(is_cat)
        def _():
            pltpu.async_copy(tab_c.at[wid >> 3, wid & 7], rowc_v, sem_c).wait()

        c_ru.wait()
        c_iu.wait()
        gather_loop(idx_u, row_v, out_u.at[wid])
        c_ri = pltpu.async_copy(tab_i.at[tf, fs], row_v, sem_r)

        @pl.when(is_cat)
        def _():
            c_ic.wait()
            gather_loop(idx_c, rowc_v, out_c.at[wid])

        @pl.when(jnp.logical_not(is_cat))
        def _():
            c_ic.wait()

        c_ri.wait()
        c_ii.wait()
        gather_loop(idx_i, row_v, out_i.at[wid])

    return _gather3


def _bn_relu(x, g, be, eps=1e-5):
    n = x.shape[0]
    ones = jnp.ones((1, n), dtype=jnp.float32)
    m = jnp.dot(ones, x, preferred_element_type=jnp.float32) / n
    e2 = jnp.dot(ones, x * x, preferred_element_type=jnp.float32) / n
    v = e2 - m * m
    a = g * jax.lax.rsqrt(v + eps)
    b = be - m * a
    return jnp.maximum(x * a + b, 0.0)


def _dot(a, b):
    bf16 = jnp.bfloat16
    return jnp.dot(a.astype(bf16), b.astype(bf16),
                   preferred_element_type=jnp.float32)


def _dot_t(a, b):
    bf16 = jnp.bfloat16
    return lax.dot_general(a.astype(bf16), b.astype(bf16),
                           (((0,), (0,)), ((), ())),
                           preferred_element_type=jnp.float32)


def _mlp_body(cont_ref, eu_ref, ei_ref, ec_ref,
              w1a_ref, w1b_ref, w1c_ref, w1d_ref, b1_ref,
              w2_ref, b2_ref, w3_ref, b3_ref, w4_ref, b4_ref,
              g1_ref, be1_ref, g2_ref, be2_ref, g3_ref, be3_ref,
              out_ref):
    h = (_dot(cont_ref[...], w1a_ref[...])
         + _dot_t(eu_ref[...], w1b_ref[...])
         + _dot_t(ei_ref[...], w1c_ref[...])
         + _dot_t(ec_ref[...], w1d_ref[...])
         + b1_ref[...])
    h = _bn_relu(h, g1_ref[...], be1_ref[...])
    h = _dot(h, w2_ref[...]) + b2_ref[...]
    h = _bn_relu(h, g2_ref[...], be2_ref[...])
    h = _dot(h, w3_ref[...]) + b3_ref[...]
    h = _bn_relu(h, g3_ref[...], be3_ref[...])
    out_ref[...] = _dot(h, w4_ref[...]) + b4_ref[...]


def kernel(continuous_features, user_id, item_id, category,
           emb_user, emb_item, emb_cat,
           W1, b1, W2, b2, W3, b3, W4, b4,
           g1, be1, g2, be2, g3, be3):
    euT, eiT, ecT = _make_gather3()(
        user_id.astype(jnp.int32), item_id.astype(jnp.int32),
        category.astype(jnp.int32),
        emb_user.T.reshape(DIM_USER // 8, 8, VOCAB_USER),
        emb_item.T.reshape(DIM_ITEM // 8, 8, VOCAB_ITEM),
        emb_cat.T.reshape(DIM_CAT // 8, 8, VOCAB_CAT))

    w1a = W1[:CONT_DIM]
    w1b = W1[CONT_DIM:CONT_DIM + DIM_USER]
    w1c = W1[CONT_DIM + DIM_USER:CONT_DIM + DIM_USER + DIM_ITEM]
    w1d = W1[CONT_DIM + DIM_USER + DIM_ITEM:]

    mlp = pl.pallas_call(
        _mlp_body,
        out_shape=jax.ShapeDtypeStruct((B, W4.shape[1]), jnp.float32),
    )
    return mlp(continuous_features, euT, eiT, ecT,
               w1a, w1b, w1c, w1d, b1.reshape(1, -1),
               W2, b2.reshape(1, -1), W3, b3.reshape(1, -1),
               W4, b4.reshape(1, -1),
               g1.reshape(1, -1), be1.reshape(1, -1),
               g2.reshape(1, -1), be2.reshape(1, -1),
               g3.reshape(1, -1), be3.reshape(1, -1))


# speedup vs baseline: 1.0640x; 1.0020x over previous
import functools

import jax
import jax.numpy as jnp
from jax import lax
from jax.experimental import pallas as pl
from jax.experimental.pallas import tpu as pltpu
from jax.experimental.pallas import tpu_sc as plsc

B = 4096
CONT_DIM = 128
DIM_USER = 32
DIM_ITEM = 32
DIM_CAT = 16
VOCAB_USER = 100000
VOCAB_ITEM = 100000
VOCAB_CAT = 1000

_NC, _NS = 2, 16
_L = 16


@functools.lru_cache(maxsize=1)
def _make_gather3():
    mesh = plsc.VectorSubcoreMesh(
        core_axis_name="c", subcore_axis_name="s",
        num_cores=_NC, num_subcores=_NS)

    @functools.partial(
        pl.kernel,
        mesh=mesh,
        out_type=(
            jax.ShapeDtypeStruct((DIM_USER, B), jnp.float32),
            jax.ShapeDtypeStruct((DIM_ITEM, B), jnp.float32),
            jax.ShapeDtypeStruct((DIM_CAT, B), jnp.float32),
        ),
        scratch_types=[
            pltpu.VMEM((B,), jnp.int32),
            pltpu.VMEM((B,), jnp.int32),
            pltpu.VMEM((B,), jnp.int32),
            pltpu.VMEM((B,), jnp.float32),
            pltpu.VMEM((VOCAB_USER,), jnp.float32),
            pltpu.VMEM((VOCAB_CAT,), jnp.float32),
            pltpu.SemaphoreType.DMA,
            pltpu.SemaphoreType.DMA,
            pltpu.SemaphoreType.DMA,
            pltpu.SemaphoreType.DMA,
            pltpu.SemaphoreType.DMA,
        ],
        compiler_params=pltpu.CompilerParams(needs_layout_passes=False),
    )
    def _gather3(uid_hbm, iid_hbm, cid_hbm, tab_u, tab_i, tab_c,
                 out_u, out_i, out_c, idx_u, idx_i, idx_c, res_v,
                 row_v, rowc_v, sem_iu, sem_ii, sem_ic, sem_r, sem_c):
        wid = lax.axis_index("s") * _NC + lax.axis_index("c")
        tf = wid >> 3
        fs = wid & 7

        def gather_loop(ids, row_buf, out_row_hbm):
            @pl.loop(0, B, step=_L, unroll=8)
            def _(c):
                iv = ids[pl.ds(c, _L)]
                res_v[pl.ds(c, _L)] = plsc.load_gather(row_buf, [iv])

            pltpu.sync_copy(res_v, out_row_hbm)

        c_iu = pltpu.async_copy(uid_hbm, idx_u, sem_iu)
        c_ii = pltpu.async_copy(iid_hbm, idx_i, sem_ii)
        c_ic = pltpu.async_copy(cid_hbm, idx_c, sem_ic)
        c_ru = pltpu.async_copy(tab_u.at[tf, fs], row_v, sem_r)
        is_cat = wid < DIM_CAT

        @pl.when(is_cat)
        def _():
            pltpu.async_copy(tab_c.at[wid >> 3, wid & 7], rowc_v, sem_c).wait()

        c_ru.wait()
        c_iu.wait()
        gather_loop(idx_u, row_v, out_u.at[wid])
        c_ri = pltpu.async_copy(tab_i.at[tf, fs], row_v, sem_r)

        @pl.when(is_cat)
        def _():
            c_ic.wait()
            gather_loop(idx_c, rowc_v, out_c.at[wid])

        @pl.when(jnp.logical_not(is_cat))
        def _():
            c_ic.wait()

        c_ri.wait()
        c_ii.wait()
        gather_loop(idx_i, row_v, out_i.at[wid])

    return _gather3


def _bn_relu(x, g, be, eps=1e-5):
    n = x.shape[0]
    ones = jnp.ones((1, n), dtype=jnp.float32)
    m = jnp.dot(ones, x, preferred_element_type=jnp.float32) / n
    e2 = jnp.dot(ones, x * x, preferred_element_type=jnp.float32) / n
    v = e2 - m * m
    a = g * jax.lax.rsqrt(v + eps)
    b = be - m * a
    return jnp.maximum(x * a + b, 0.0)


def _dot(a, b):
    bf16 = jnp.bfloat16
    return jnp.dot(a.astype(bf16), b.astype(bf16),
                   preferred_element_type=jnp.float32)


def _dot_t(a, b):
    bf16 = jnp.bfloat16
    return lax.dot_general(a.astype(bf16), b.astype(bf16),
                           (((0,), (0,)), ((), ())),
                           preferred_element_type=jnp.float32)


def _mlp_body(cont_ref, eu_ref, ei_ref, ec_ref,
              w1a_ref, w1b_ref, w1c_ref, w1d_ref, b1_ref,
              w2_ref, b2_ref, w3_ref, b3_ref, w4_ref, b4_ref,
              g1_ref, be1_ref, g2_ref, be2_ref, g3_ref, be3_ref,
              out_ref):
    h = (_dot(cont_ref[...], w1a_ref[...])
         + _dot_t(eu_ref[...], w1b_ref[...])
         + _dot_t(ei_ref[...], w1c_ref[...])
         + _dot_t(ec_ref[...], w1d_ref[...])
         + b1_ref[...])
    h = _bn_relu(h, g1_ref[...], be1_ref[...])
    h = _dot(h, w2_ref[...]) + b2_ref[...]
    h = _bn_relu(h, g2_ref[...], be2_ref[...])
    h = _dot(h, w3_ref[...]) + b3_ref[...]
    h = _bn_relu(h, g3_ref[...], be3_ref[...])
    out_ref[...] = _dot(h, w4_ref[...]) + b4_ref[...]


def kernel(continuous_features, user_id, item_id, category,
           emb_user, emb_item, emb_cat,
           W1, b1, W2, b2, W3, b3, W4, b4,
           g1, be1, g2, be2, g3, be3):
    euT, eiT, ecT = _make_gather3()(
        user_id.astype(jnp.int32), item_id.astype(jnp.int32),
        category.astype(jnp.int32),
        emb_user.T.reshape(DIM_USER // 8, 8, VOCAB_USER),
        emb_item.T.reshape(DIM_ITEM // 8, 8, VOCAB_ITEM),
        emb_cat.T.reshape(DIM_CAT // 8, 8, VOCAB_CAT))

    w1a = W1[:CONT_DIM]
    w1b = W1[CONT_DIM:CONT_DIM + DIM_USER]
    w1c = W1[CONT_DIM + DIM_USER:CONT_DIM + DIM_USER + DIM_ITEM]
    w1d = W1[CONT_DIM + DIM_USER + DIM_ITEM:]

    mlp = pl.pallas_call(
        _mlp_body,
        out_shape=jax.ShapeDtypeStruct((B, W4.shape[1]), jnp.float32),
    )
    return mlp(continuous_features, euT, eiT, ecT,
               w1a, w1b, w1c, w1d, b1.reshape(1, -1),
               W2, b2.reshape(1, -1), W3, b3.reshape(1, -1),
               W4, b4.reshape(1, -1),
               g1.reshape(1, -1), be1.reshape(1, -1),
               g2.reshape(1, -1), be2.reshape(1, -1),
               g3.reshape(1, -1), be3.reshape(1, -1))
